# baseline (speedup 1.0000x reference)
import functools

import jax
import jax.numpy as jnp
from jax import lax
from jax.experimental import pallas as pl
from jax.experimental.pallas import tpu as pltpu
from jax.experimental.pallas import tpu_sc as plsc

_NC = 2
_NS = 16
_NW = _NC * _NS
_CH = 64
_NBUF = 4
_D = 128


def _partial16(ri, rj, q):
    prods = [ri[q, pl.ds(k * 16, 16)] * rj[q, pl.ds(k * 16, 16)]
             for k in range(_D // 16)]
    while len(prods) > 1:
        prods = [x + y for x, y in zip(prods[::2], prods[1::2])]
    return prods[0]


def _sc_body(ppw, nchunk, vi_hbm, vj_hbm, node_hbm, ctx_hbm, out_hbm,
             idx_i, idx_j,
             rows_i0, rows_j0, rows_i1, rows_j1,
             rows_i2, rows_j2, rows_i3, rows_j3, out_v,
             part_v0, part_v1, part_v2, part_v3,
             sem_ii, sem_ij, sem_i0, sem_j0, sem_i1, sem_j1,
             sem_i2, sem_j2, sem_i3, sem_j3, sem_out):
    wid = lax.axis_index("s") * _NC + lax.axis_index("c")
    base = wid * ppw
    lane = lax.iota(jnp.int32, 16)

    bufs = ((rows_i0, rows_j0, sem_i0, sem_j0),
            (rows_i1, rows_j1, sem_i1, sem_j1),
            (rows_i2, rows_j2, sem_i2, sem_j2),
            (rows_i3, rows_j3, sem_i3, sem_j3))
    parts = (part_v0, part_v1, part_v2, part_v3)

    cpi = pltpu.async_copy(vi_hbm.at[pl.ds(base, ppw)], idx_i, sem_ii)
    cpj = pltpu.async_copy(vj_hbm.at[pl.ds(base, ppw)], idx_j, sem_ij)

    cpi.wait()
    cpj.wait()

    def start(c, parity):
        ri, rj, si, sj = bufs[parity]
        sl = pl.ds(c * _CH, _CH)
        pltpu.async_copy(node_hbm.at[idx_i.at[sl]], ri, si)
        pltpu.async_copy(ctx_hbm.at[idx_j.at[sl]], rj, sj)

    def drain(parity):
        ri, rj, si, sj = bufs[parity]
        pltpu.make_async_copy(node_hbm.at[idx_i.at[pl.ds(0, _CH)]], ri, si).wait()
        pltpu.make_async_copy(ctx_hbm.at[idx_j.at[pl.ds(0, _CH)]], rj, sj).wait()

    def compute(c, parity):
        ri, rj, _, _ = bufs[parity]
        part = parts[parity]

        def grp_body(g, carry):
            for p in range(16):
                q = g * 16 + p
                part[q, pl.ds(0, 16)] = _partial16(ri, rj, q)
            return carry

        lax.fori_loop(0, _CH // 16, grp_body, 0)

        def red_body(g, carry):
            rows = g * 16 + lane
            cols = [plsc.load_gather(part, [rows, jnp.full((16,), t, jnp.int32)])
                    for t in range(16)]
            while len(cols) > 1:
                cols = [x + y for x, y in zip(cols[::2], cols[1::2])]
            out_v[pl.ds(c * _CH + g * 16, 16)] = cols[0]
            return carry

        lax.fori_loop(0, _CH // 16, red_body, 0)

    for s in range(_NBUF):
        start(s, s)

    def ring_body(i, carry):
        c0 = i * _NBUF
        for s in range(_NBUF):
            c = c0 + s
            drain(s)
            compute(c, s)

            @pl.when(c + _NBUF < nchunk)
            def _():
                start(c + _NBUF, s)

        return carry

    lax.fori_loop(0, nchunk // _NBUF, ring_body, 0)

    pltpu.async_copy(out_v, out_hbm.at[pl.ds(base, ppw)], sem_out).wait()


def _sc_inner(vi, vj, node_tab, ctx_tab):
    n = vi.shape[0]
    ppw = n // _NW
    nchunk = ppw // _CH
    mesh = plsc.VectorSubcoreMesh(
        core_axis_name="c", subcore_axis_name="s",
        num_cores=_NC, num_subcores=_NS)
    k = pl.kernel(
        functools.partial(_sc_body, ppw, nchunk),
        out_type=jax.ShapeDtypeStruct((n,), jnp.float32),
        mesh=mesh,
        compiler_params=pltpu.CompilerParams(needs_layout_passes=False),
        scratch_types=[
            pltpu.VMEM((ppw,), jnp.int32),
            pltpu.VMEM((ppw,), jnp.int32),
        ] + [pltpu.VMEM((_CH, _D), jnp.float32)] * (2 * _NBUF) + [
            pltpu.VMEM((ppw,), jnp.float32),
        ] + [pltpu.VMEM((_CH, 24), jnp.float32)] * _NBUF
          + [pltpu.SemaphoreType.DMA] * (3 + 2 * _NBUF),
    )
    return k(vi, vj, node_tab, ctx_tab)


def _tc_loss_body(inv_b, x_ref, l_ref, o_ref):
    z = l_ref[...] * x_ref[...]
    ls = jnp.minimum(z, 0.0) - jnp.log1p(jnp.exp(-jnp.abs(z)))
    o_ref[0, 0] = -jnp.sum(ls) * inv_b


def _tc_loss(inner2d, labels2d, inv_b):
    out = pl.pallas_call(
        functools.partial(_tc_loss_body, inv_b),
        out_shape=jax.ShapeDtypeStruct((1, 1), jnp.float32),
        out_specs=pl.BlockSpec(memory_space=pltpu.SMEM),
    )(inner2d, labels2d)
    return out[0, 0]


def kernel(v_i, v_j, labels, batch_size, node_embeddings, contextnode_embeddings):
    b, k = v_i.shape
    n = b * k
    vi = v_i.reshape(n).astype(jnp.int32)
    vj = v_j.reshape(n).astype(jnp.int32)
    inner = _sc_inner(vi, vj, node_embeddings, contextnode_embeddings)
    rows = n // 128
    return _tc_loss(inner.reshape(rows, 128), labels.reshape(rows, 128),
                    1.0 / float(b))


# speedup vs baseline: 1.1498x; 1.1498x over previous
import functools

import jax
import jax.numpy as jnp
from jax import lax
from jax.experimental import pallas as pl
from jax.experimental.pallas import tpu as pltpu
from jax.experimental.pallas import tpu_sc as plsc

_NC = 2
_NS = 16
_NW = _NC * _NS
_CH = 128
_NBUF = 2
_D = 128


def _partial16(ri, rj, q):
    prods = [ri[q, pl.ds(k * 16, 16)] * rj[q, pl.ds(k * 16, 16)]
             for k in range(_D // 16)]
    while len(prods) > 1:
        prods = [x + y for x, y in zip(prods[::2], prods[1::2])]
    return prods[0]


def _sc_body(ppw, nchunk, vi_hbm, vj_hbm, node_hbm, ctx_hbm, out_hbm,
             idx_i, idx_j,
             rows_i0, rows_j0, rows_i1, rows_j1, out_v,
             part_v0, part_v1,
             sem_ii, sem_ij, sem_i0, sem_j0, sem_i1, sem_j1, sem_out):
    wid = lax.axis_index("s") * _NC + lax.axis_index("c")
    base = wid * ppw
    lane = lax.iota(jnp.int32, 16)

    bufs = ((rows_i0, rows_j0, sem_i0, sem_j0),
            (rows_i1, rows_j1, sem_i1, sem_j1))
    parts = (part_v0, part_v1)

    cpi = pltpu.async_copy(vi_hbm.at[pl.ds(base, ppw)], idx_i, sem_ii)
    cpj = pltpu.async_copy(vj_hbm.at[pl.ds(base, ppw)], idx_j, sem_ij)

    cpi.wait()
    cpj.wait()

    def start(c, parity):
        ri, rj, si, sj = bufs[parity]
        sl = pl.ds(c * _CH, _CH)
        pltpu.async_copy(node_hbm.at[idx_i.at[sl]], ri, si)
        pltpu.async_copy(ctx_hbm.at[idx_j.at[sl]], rj, sj)

    def drain(parity):
        ri, rj, si, sj = bufs[parity]
        pltpu.make_async_copy(node_hbm.at[idx_i.at[pl.ds(0, _CH)]], ri, si).wait()
        pltpu.make_async_copy(ctx_hbm.at[idx_j.at[pl.ds(0, _CH)]], rj, sj).wait()

    def compute(c, parity):
        ri, rj, _, _ = bufs[parity]
        part = parts[parity]

        def grp_body(g, carry):
            for p in range(16):
                q = g * 16 + p
                part[q, pl.ds(0, 16)] = _partial16(ri, rj, q)
            return carry

        lax.fori_loop(0, _CH // 16, grp_body, 0)

        def red_body(g, carry):
            rows = g * 16 + lane
            cols = [plsc.load_gather(part, [rows, jnp.full((16,), t, jnp.int32)])
                    for t in range(16)]
            while len(cols) > 1:
                cols = [x + y for x, y in zip(cols[::2], cols[1::2])]
            out_v[pl.ds(c * _CH + g * 16, 16)] = cols[0]
            return carry

        lax.fori_loop(0, _CH // 16, red_body, 0)

    for s in range(_NBUF):
        start(s, s)

    def ring_body(i, carry):
        c0 = i * _NBUF
        for s in range(_NBUF):
            c = c0 + s
            drain(s)
            compute(c, s)

            @pl.when(c + _NBUF < nchunk)
            def _():
                start(c + _NBUF, s)

        return carry

    lax.fori_loop(0, nchunk // _NBUF, ring_body, 0)

    pltpu.async_copy(out_v, out_hbm.at[pl.ds(base, ppw)], sem_out).wait()


def _sc_inner(vi, vj, node_tab, ctx_tab):
    n = vi.shape[0]
    ppw = n // _NW
    nchunk = ppw // _CH
    mesh = plsc.VectorSubcoreMesh(
        core_axis_name="c", subcore_axis_name="s",
        num_cores=_NC, num_subcores=_NS)
    k = pl.kernel(
        functools.partial(_sc_body, ppw, nchunk),
        out_type=jax.ShapeDtypeStruct((n,), jnp.float32),
        mesh=mesh,
        compiler_params=pltpu.CompilerParams(needs_layout_passes=False),
        scratch_types=[
            pltpu.VMEM((ppw,), jnp.int32),
            pltpu.VMEM((ppw,), jnp.int32),
        ] + [pltpu.VMEM((_CH, _D), jnp.float32)] * (2 * _NBUF) + [
            pltpu.VMEM((ppw,), jnp.float32),
        ] + [pltpu.VMEM((_CH, 24), jnp.float32)] * _NBUF
          + [pltpu.SemaphoreType.DMA] * (3 + 2 * _NBUF),
    )
    return k(vi, vj, node_tab, ctx_tab)


def _tc_loss_body(inv_b, x_ref, l_ref, o_ref):
    z = l_ref[...] * x_ref[...]
    ls = jnp.minimum(z, 0.0) - jnp.log1p(jnp.exp(-jnp.abs(z)))
    o_ref[0, 0] = -jnp.sum(ls) * inv_b


def _tc_loss(inner2d, labels2d, inv_b):
    out = pl.pallas_call(
        functools.partial(_tc_loss_body, inv_b),
        out_shape=jax.ShapeDtypeStruct((1, 1), jnp.float32),
        out_specs=pl.BlockSpec(memory_space=pltpu.SMEM),
    )(inner2d, labels2d)
    return out[0, 0]


def kernel(v_i, v_j, labels, batch_size, node_embeddings, contextnode_embeddings):
    b, k = v_i.shape
    n = b * k
    vi = v_i.reshape(n).astype(jnp.int32)
    vj = v_j.reshape(n).astype(jnp.int32)
    inner = _sc_inner(vi, vj, node_embeddings, contextnode_embeddings)
    rows = n // 128
    return _tc_loss(inner.reshape(rows, 128), labels.reshape(rows, 128),
                    1.0 / float(b))


# speedup vs baseline: 1.1827x; 1.0286x over previous
import functools

import jax
import jax.numpy as jnp
from jax import lax
from jax.experimental import pallas as pl
from jax.experimental.pallas import tpu as pltpu
from jax.experimental.pallas import tpu_sc as plsc

_NC = 2
_NS = 16
_NW = _NC * _NS
_CH = 128
_NBUF = 2
_D = 128


def _partial16(ri, rj, q):
    prods = [ri[q, pl.ds(k * 16, 16)] * rj[q, pl.ds(k * 16, 16)]
             for k in range(_D // 16)]
    while len(prods) > 1:
        prods = [x + y for x, y in zip(prods[::2], prods[1::2])]
    return prods[0]


def _sc_body(ppw, nchunk, n, vij_hbm, node_hbm, ctx_hbm, out_hbm,
             idx_i, idx_j,
             rows_i0, rows_j0, rows_i1, rows_j1, out_v,
             part_v0, part_v1,
             sem_ii, sem_ij, sem_i0, sem_j0, sem_i1, sem_j1, sem_out):
    wid = lax.axis_index("s") * _NC + lax.axis_index("c")
    base = wid * ppw
    lane = lax.iota(jnp.int32, 16)

    bufs = ((rows_i0, rows_j0, sem_i0, sem_j0),
            (rows_i1, rows_j1, sem_i1, sem_j1))
    parts = (part_v0, part_v1)

    cpi = pltpu.async_copy(vij_hbm.at[pl.ds(base, ppw)], idx_i, sem_ii)
    cpj = pltpu.async_copy(vij_hbm.at[pl.ds(n + base, ppw)], idx_j, sem_ij)

    cpi.wait()
    cpj.wait()

    def start(c, parity):
        ri, rj, si, sj = bufs[parity]
        sl = pl.ds(c * _CH, _CH)
        pltpu.async_copy(node_hbm.at[idx_i.at[sl]], ri, si)
        pltpu.async_copy(ctx_hbm.at[idx_j.at[sl]], rj, sj)

    def drain(parity):
        ri, rj, si, sj = bufs[parity]
        pltpu.make_async_copy(node_hbm.at[idx_i.at[pl.ds(0, _CH)]], ri, si).wait()
        pltpu.make_async_copy(ctx_hbm.at[idx_j.at[pl.ds(0, _CH)]], rj, sj).wait()

    def compute(c, parity):
        ri, rj, _, _ = bufs[parity]
        part = parts[parity]

        def grp_body(g, carry):
            for p in range(16):
                q = g * 16 + p
                part[q, pl.ds(0, 16)] = _partial16(ri, rj, q)
            return carry

        lax.fori_loop(0, _CH // 16, grp_body, 0)

        def red_body(g, carry):
            rows = g * 16 + lane
            cols = [plsc.load_gather(part, [rows, jnp.full((16,), t, jnp.int32)])
                    for t in range(16)]
            while len(cols) > 1:
                cols = [x + y for x, y in zip(cols[::2], cols[1::2])]
            out_v[pl.ds(c * _CH + g * 16, 16)] = cols[0]
            return carry

        lax.fori_loop(0, _CH // 16, red_body, 0)

    for s in range(_NBUF):
        start(s, s)

    def ring_body(i, carry):
        c0 = i * _NBUF
        for s in range(_NBUF):
            c = c0 + s
            drain(s)
            compute(c, s)

            @pl.when(c + _NBUF < nchunk)
            def _():
                start(c + _NBUF, s)

        return carry

    lax.fori_loop(0, nchunk // _NBUF, ring_body, 0)

    pltpu.async_copy(out_v, out_hbm.at[pl.ds(base, ppw)], sem_out).wait()


def _sc_inner(vij, node_tab, ctx_tab):
    n = vij.shape[0] // 2
    ppw = n // _NW
    nchunk = ppw // _CH
    mesh = plsc.VectorSubcoreMesh(
        core_axis_name="c", subcore_axis_name="s",
        num_cores=_NC, num_subcores=_NS)
    k = pl.kernel(
        functools.partial(_sc_body, ppw, nchunk, n),
        out_type=jax.ShapeDtypeStruct((n,), jnp.float32),
        mesh=mesh,
        compiler_params=pltpu.CompilerParams(needs_layout_passes=False),
        scratch_types=[
            pltpu.VMEM((ppw,), jnp.int32),
            pltpu.VMEM((ppw,), jnp.int32),
        ] + [pltpu.VMEM((_CH, _D), jnp.float32)] * (2 * _NBUF) + [
            pltpu.VMEM((ppw,), jnp.float32),
        ] + [pltpu.VMEM((_CH, 24), jnp.float32)] * _NBUF
          + [pltpu.SemaphoreType.DMA] * (3 + 2 * _NBUF),
    )
    return k(vij, node_tab, ctx_tab)


def _tc_loss_body(inv_b, x_ref, l_ref, o_ref):
    z = l_ref[...] * x_ref[...]
    ls = jnp.minimum(z, 0.0) - jnp.log1p(jnp.exp(-jnp.abs(z)))
    o_ref[0, 0] = -jnp.sum(ls) * inv_b


def _tc_loss(inner2d, labels2d, inv_b):
    out = pl.pallas_call(
        functools.partial(_tc_loss_body, inv_b),
        out_shape=jax.ShapeDtypeStruct((1, 1), jnp.float32),
        out_specs=pl.BlockSpec(memory_space=pltpu.SMEM),
    )(inner2d, labels2d)
    return out[0, 0]


def kernel(v_i, v_j, labels, batch_size, node_embeddings, contextnode_embeddings):
    b, k = v_i.shape
    n = b * k
    vij = jnp.stack([v_i.astype(jnp.int32),
                     v_j.astype(jnp.int32)]).reshape(2 * n)
    inner = _sc_inner(vij, node_embeddings, contextnode_embeddings)
    rows = n // 128
    return _tc_loss(inner.reshape(rows, 128), labels.reshape(rows, 128),
                    1.0 / float(b))
